# baseline (speedup 1.0000x reference)
import functools

import jax
import jax.numpy as jnp
from jax.experimental import pallas as pl
from jax.experimental.pallas import tpu as pltpu

_CHUNK = 40
_NBUF = 16


def _gcn_kernel(x_ref, w_ref, b_ref, adj_hbm, o_ref, s_ref, buf_ref, sem,
                *, chunk, nbuf, nchunks):
    def start(i):
        slot = jax.lax.rem(i, nbuf)
        pltpu.make_async_copy(
            adj_hbm.at[pl.ds(i * chunk, chunk), :],
            buf_ref.at[slot],
            sem.at[slot],
        ).start()

    for i in range(nbuf):
        start(i)

    s_ref[...] = jnp.dot(x_ref[...], w_ref[...],
                         preferred_element_type=jnp.float32)

    def body(i, carry):
        slot = jax.lax.rem(i, nbuf)
        pltpu.make_async_copy(
            adj_hbm.at[pl.ds(i * chunk, chunk), :],
            buf_ref.at[slot],
            sem.at[slot],
        ).wait()
        o_ref[pl.ds(i * chunk, chunk), :] = jnp.dot(
            buf_ref[slot], s_ref[...],
            preferred_element_type=jnp.float32) + b_ref[...]

        @pl.when(i + nbuf < nchunks)
        def _():
            start(i + nbuf)

        return carry

    jax.lax.fori_loop(0, nchunks, body, 0)


def kernel(x, adj, W, b):
    n, nfeat = x.shape
    nclass = W.shape[1]
    nchunks = n // _CHUNK

    out = pl.pallas_call(
        functools.partial(_gcn_kernel, chunk=_CHUNK, nbuf=_NBUF,
                          nchunks=nchunks),
        in_specs=[
            pl.BlockSpec((n, nfeat), lambda: (0, 0)),
            pl.BlockSpec((nfeat, nclass), lambda: (0, 0)),
            pl.BlockSpec((1, nclass), lambda: (0, 0)),
            pl.BlockSpec(memory_space=pl.ANY),
        ],
        out_specs=pl.BlockSpec((n, nclass), lambda: (0, 0)),
        out_shape=jax.ShapeDtypeStruct((n, nclass), jnp.float32),
        scratch_shapes=[
            pltpu.VMEM((n, nclass), jnp.float32),
            pltpu.VMEM((_NBUF, _CHUNK, n), jnp.float32),
            pltpu.SemaphoreType.DMA((_NBUF,)),
        ],
    )(x, W, b.reshape(1, nclass), adj)
    return out


# speedup vs baseline: 1.1710x; 1.1710x over previous
import functools

import jax
import jax.numpy as jnp
from jax.experimental import pallas as pl
from jax.experimental.pallas import tpu as pltpu

_SUB = 40
_SPM = 5
_NSLOT = 20


def _gcn_kernel(x_ref, w_ref, b_ref, adj_hbm, o_ref, s_ref, buf_ref, sem,
                *, n, nsub, nmacro):
    def copy(j):
        slot = jax.lax.rem(j, _NSLOT)
        return pltpu.make_async_copy(
            adj_hbm.at[pl.ds(j * _SUB, _SUB), :],
            buf_ref.at[pl.ds(slot * _SUB, _SUB), :],
            sem.at[slot],
        )

    for j in range(_NSLOT):
        copy(j).start()

    s_ref[...] = jnp.dot(x_ref[...], w_ref[...],
                         preferred_element_type=jnp.float32)

    rows = _SPM * _SUB

    def body(m, carry):
        j0 = m * _SPM
        for t in range(_SPM):
            copy(j0 + t).wait()
        base = jax.lax.rem(j0, _NSLOT) * _SUB
        o_ref[pl.ds(m * rows, rows), :] = jnp.dot(
            buf_ref[pl.ds(base, rows), :], s_ref[...],
            preferred_element_type=jnp.float32) + b_ref[...]
        for t in range(_SPM):
            j = j0 + _NSLOT + t

            @pl.when(j < nsub)
            def _():
                copy(j).start()

        return carry

    jax.lax.fori_loop(0, nmacro, body, 0)


def kernel(x, adj, W, b):
    n, nfeat = x.shape
    nclass = W.shape[1]
    nsub = n // _SUB
    nmacro = nsub // _SPM

    out = pl.pallas_call(
        functools.partial(_gcn_kernel, n=n, nsub=nsub, nmacro=nmacro),
        in_specs=[
            pl.BlockSpec((n, nfeat), lambda: (0, 0)),
            pl.BlockSpec((nfeat, nclass), lambda: (0, 0)),
            pl.BlockSpec((1, nclass), lambda: (0, 0)),
            pl.BlockSpec(memory_space=pl.ANY),
        ],
        out_specs=pl.BlockSpec((n, nclass), lambda: (0, 0)),
        out_shape=jax.ShapeDtypeStruct((n, nclass), jnp.float32),
        scratch_shapes=[
            pltpu.VMEM((n, nclass), jnp.float32),
            pltpu.VMEM((_NSLOT * _SUB, n), jnp.float32),
            pltpu.SemaphoreType.DMA((_NSLOT,)),
        ],
    )(x, W, b.reshape(1, nclass), adj)
    return out
